# Initial kernel scaffold; baseline (speedup 1.0000x reference)
import jax
import jax.numpy as jnp
from jax.experimental import pallas as pl


def kernel(x, deg, scale):
    raise NotImplementedError("write your pallas kernel here")


# speedup vs baseline: 2.9265x; 2.9265x over previous
import jax
import jax.numpy as jnp
from jax.experimental import pallas as pl

_BLOCK_ROWS = 1000


def _body(deg_ref, scale_ref, x_ref, out_ref):
    s = jnp.exp(scale_ref[...])
    d = deg_ref[0, 0, :]
    iota = jax.lax.broadcasted_iota(jnp.int32, (1, 4), 1)
    onehot = (d[:, None] == iota).astype(jnp.float32)
    m = jnp.dot(onehot, s, preferred_element_type=jnp.float32)
    out_ref[...] = m * x_ref[...]


def kernel(x, deg, scale):
    n, w = x.shape
    b = _BLOCK_ROWS
    while n % b:
        b //= 2
    nb = n // b
    deg3 = deg.astype(jnp.int32).reshape(nb, 1, b)
    return pl.pallas_call(
        _body,
        grid=(nb,),
        in_specs=[
            pl.BlockSpec((1, 1, b), lambda i: (i, 0, 0)),
            pl.BlockSpec((4, w), lambda i: (0, 0)),
            pl.BlockSpec((b, w), lambda i: (i, 0)),
        ],
        out_specs=pl.BlockSpec((b, w), lambda i: (i, 0)),
        out_shape=jax.ShapeDtypeStruct((n, w), x.dtype),
    )(deg3, scale, x)


# speedup vs baseline: 4.7600x; 1.6265x over previous
import jax
import jax.numpy as jnp
from jax.experimental import pallas as pl

_BLOCK_ROWS = 4000


def _body(deg_ref, scale_ref, x_ref, out_ref):
    s = jnp.exp(scale_ref[...])
    d = deg_ref[0, 0, :]
    iota = jax.lax.broadcasted_iota(jnp.int32, (1, 4), 1)
    onehot = (d[:, None] == iota).astype(jnp.float32)
    m = jnp.dot(onehot, s, preferred_element_type=jnp.float32)
    out_ref[...] = m * x_ref[...]


def kernel(x, deg, scale):
    n, w = x.shape
    b = _BLOCK_ROWS
    while n % b:
        b //= 2
    nb = n // b
    deg3 = deg.astype(jnp.int32).reshape(nb, 1, b)
    return pl.pallas_call(
        _body,
        grid=(nb,),
        in_specs=[
            pl.BlockSpec((1, 1, b), lambda i: (i, 0, 0)),
            pl.BlockSpec((4, w), lambda i: (0, 0)),
            pl.BlockSpec((b, w), lambda i: (i, 0)),
        ],
        out_specs=pl.BlockSpec((b, w), lambda i: (i, 0)),
        out_shape=jax.ShapeDtypeStruct((n, w), x.dtype),
    )(deg3, scale, x)


# speedup vs baseline: 4.8294x; 1.0146x over previous
import jax
import jax.numpy as jnp
from jax.experimental import pallas as pl

_BLOCK_ROWS = 10000


def _body(deg_ref, scale_ref, x_ref, out_ref):
    s = jnp.exp(scale_ref[...])
    d = deg_ref[0, 0, :]
    iota = jax.lax.broadcasted_iota(jnp.int32, (1, 4), 1)
    onehot = (d[:, None] == iota).astype(jnp.float32)
    m = jnp.dot(onehot, s, preferred_element_type=jnp.float32)
    out_ref[...] = m * x_ref[...]


def kernel(x, deg, scale):
    n, w = x.shape
    b = _BLOCK_ROWS
    while n % b:
        b //= 2
    nb = n // b
    deg3 = deg.astype(jnp.int32).reshape(nb, 1, b)
    return pl.pallas_call(
        _body,
        grid=(nb,),
        in_specs=[
            pl.BlockSpec((1, 1, b), lambda i: (i, 0, 0)),
            pl.BlockSpec((4, w), lambda i: (0, 0)),
            pl.BlockSpec((b, w), lambda i: (i, 0)),
        ],
        out_specs=pl.BlockSpec((b, w), lambda i: (i, 0)),
        out_shape=jax.ShapeDtypeStruct((n, w), x.dtype),
    )(deg3, scale, x)
